# baseline (speedup 1.0000x reference)
import functools

import jax
import jax.numpy as jnp
from jax.experimental import pallas as pl
from jax.experimental.pallas import tpu as pltpu

_EPS = 1e-5


def _stats_fold(x_ref, w_ref, b_ref, wf_ref, bf_ref, acc_ref, *, nsteps, inv_n):
    i = pl.program_id(0)

    @pl.when(i == 0)
    def _():
        acc_ref[...] = jnp.zeros_like(acc_ref)

    xb = x_ref[...]
    acc_ref[0:1, :] += jnp.sum(xb, axis=0, keepdims=True)
    acc_ref[1:2, :] += jnp.sum(xb * xb, axis=0, keepdims=True)

    @pl.when(i == nsteps - 1)
    def _():
        mean = acc_ref[0:1, :] * inv_n
        var = acc_ref[1:2, :] * inv_n - mean * mean
        rstd = jax.lax.rsqrt(var + _EPS)
        wf = w_ref[...] * rstd
        wf_ref[...] = wf.astype(jnp.bfloat16)
        bf_ref[...] = b_ref[...] - jax.lax.dot_general(
            mean, wf, (((1,), (1,)), ((), ())),
            preferred_element_type=jnp.float32)


def _mm(x_ref, wf_ref, bf_ref, o_ref):
    o_ref[...] = jax.lax.dot_general(
        x_ref[...].astype(jnp.bfloat16), wf_ref[...],
        (((1,), (1,)), ((), ())),
        preferred_element_type=jnp.float32) + bf_ref[...]


def kernel(nodeblocks, x, W, b):
    n, d = x.shape
    c = W.shape[0]
    blk = 25000
    nb = n // blk
    b2 = b.reshape(1, c)

    wf, bf = pl.pallas_call(
        functools.partial(_stats_fold, nsteps=nb, inv_n=1.0 / n),
        grid=(nb,),
        in_specs=[
            pl.BlockSpec((blk, d), lambda i: (i, 0)),
            pl.BlockSpec((c, d), lambda i: (0, 0)),
            pl.BlockSpec((1, c), lambda i: (0, 0)),
        ],
        out_specs=[
            pl.BlockSpec((c, d), lambda i: (0, 0)),
            pl.BlockSpec((1, c), lambda i: (0, 0)),
        ],
        out_shape=[
            jax.ShapeDtypeStruct((c, d), jnp.bfloat16),
            jax.ShapeDtypeStruct((1, c), jnp.float32),
        ],
        scratch_shapes=[pltpu.VMEM((2, d), jnp.float32)],
    )(x, W, b2)

    out = pl.pallas_call(
        _mm,
        grid=(nb,),
        in_specs=[
            pl.BlockSpec((blk, d), lambda i: (i, 0)),
            pl.BlockSpec((c, d), lambda i: (0, 0)),
            pl.BlockSpec((1, c), lambda i: (0, 0)),
        ],
        out_specs=pl.BlockSpec((blk, c), lambda i: (i, 0)),
        out_shape=jax.ShapeDtypeStruct((n, c), jnp.float32),
        compiler_params=pltpu.CompilerParams(
            dimension_semantics=("parallel",)),
    )(x, wf, bf)
    return out


# speedup vs baseline: 1.1540x; 1.1540x over previous
import functools

import jax
import jax.numpy as jnp
from jax.experimental import pallas as pl
from jax.experimental.pallas import tpu as pltpu

_EPS = 1e-5


def _fused(x_ref, w_ref, b_ref, o_ref, xbuf_ref, acc_ref, wf_ref, bf_ref,
           *, nb, blk, inv_n):
    i = pl.program_id(0)

    @pl.when(i == 0)
    def _():
        acc_ref[...] = jnp.zeros_like(acc_ref)

    @pl.when(i < nb)
    def _():
        xb = x_ref[...]
        acc_ref[0:1, :] += jnp.sum(xb, axis=0, keepdims=True)
        acc_ref[1:2, :] += jnp.sum(xb * xb, axis=0, keepdims=True)
        xbuf_ref[pl.ds(i * blk, blk), :] = xb.astype(jnp.bfloat16)

    @pl.when(i == nb)
    def _():
        mean = acc_ref[0:1, :] * inv_n
        var = acc_ref[1:2, :] * inv_n - mean * mean
        rstd = jax.lax.rsqrt(var + _EPS)
        wf = w_ref[...] * rstd
        wf_ref[...] = wf.astype(jnp.bfloat16)
        bf_ref[...] = b_ref[...] - jax.lax.dot_general(
            mean, wf, (((1,), (1,)), ((), ())),
            preferred_element_type=jnp.float32)

    @pl.when(i >= nb)
    def _():
        j = i - nb
        xb = xbuf_ref[pl.ds(j * blk, blk), :]
        o_ref[...] = jax.lax.dot_general(
            xb, wf_ref[...], (((1,), (1,)), ((), ())),
            preferred_element_type=jnp.float32) + bf_ref[...]


def kernel(nodeblocks, x, W, b):
    n, d = x.shape
    c = W.shape[0]
    blk = 10000
    nb = n // blk
    b2 = b.reshape(1, c)

    out = pl.pallas_call(
        functools.partial(_fused, nb=nb, blk=blk, inv_n=1.0 / n),
        grid=(2 * nb,),
        in_specs=[
            pl.BlockSpec((blk, d), lambda i: (jnp.minimum(i, nb - 1), 0)),
            pl.BlockSpec((c, d), lambda i: (0, 0)),
            pl.BlockSpec((1, c), lambda i: (0, 0)),
        ],
        out_specs=pl.BlockSpec((blk, c), lambda i: (jnp.maximum(i - nb, 0), 0)),
        out_shape=jax.ShapeDtypeStruct((n, c), jnp.float32),
        scratch_shapes=[
            pltpu.VMEM((n, d), jnp.bfloat16),
            pltpu.VMEM((2, d), jnp.float32),
            pltpu.VMEM((c, d), jnp.bfloat16),
            pltpu.VMEM((1, c), jnp.float32),
        ],
    )(x, W, b2)
    return out
